# baseline (speedup 1.0000x reference)
import jax
import jax.numpy as jnp
from jax.experimental import pallas as pl
from jax.experimental.pallas import tpu as pltpu
from jax.experimental.pallas import tpu_sc as plsc
from jax.scipy.linalg import block_diag

N = 2048
D = 1024
H = 8
HD = D // H
E = 8
I = 2 * HD
TB = 512
SCL = 16


def _head_router_block(x_ref, w_hpt_ref, b_hp_ref, embbd_ref, h_ref, lt_ref):
    x = x_ref[...].astype(jnp.bfloat16)
    h = jnp.dot(x, w_hpt_ref[...], preferred_element_type=jnp.float32)
    h = h + b_hp_ref[...]
    h_ref[...] = h.astype(jnp.bfloat16)
    logits = jnp.dot(h, embbd_ref[...],
                     preferred_element_type=jnp.float32)
    lt_ref[...] = logits.T


def _router_sc(logitsT):
    mesh = plsc.VectorSubcoreMesh(core_axis_name="core",
                                  subcore_axis_name="subcore")

    @pl.kernel(out_type=jax.ShapeDtypeStruct((H * E, N), jnp.float32),
               mesh=mesh,
               compiler_params=pltpu.CompilerParams(
                   use_tc_tiling_on_sc=False))
    def run(l_hbm, o_hbm):
        def body(l_vmem, o_vmem):
            for hh in range(H):
                vs = [l_vmem[hh * E + e] for e in range(E)]
                m = vs[0]
                for e in range(1, E):
                    m = jnp.maximum(m, vs[e])
                exs = [jnp.exp(v - m) for v in vs]
                z = exs[0]
                for e in range(1, E):
                    z = z + exs[e]
                inv = 1.0 / z
                ps = [ex * inv for ex in exs]
                b1 = ps[0]
                i1 = jnp.zeros((SCL,), jnp.int32)
                for e in range(1, E):
                    c = ps[e] > b1
                    b1 = jnp.where(c, ps[e], b1)
                    i1 = jnp.where(c, e, i1)
                b2 = jnp.full((SCL,), -1.0, jnp.float32)
                i2 = jnp.zeros((SCL,), jnp.int32)
                for e in range(E):
                    cand = jnp.where(i1 == e, -1.0, ps[e])
                    c = cand > b2
                    b2 = jnp.where(c, cand, b2)
                    i2 = jnp.where(c, e, i2)
                for e in range(E):
                    w_e = (jnp.where(i1 == e, b1, 0.0)
                           + jnp.where(i2 == e, b2, 0.0))
                    o_vmem[hh * E + e] = w_e

        pltpu.emit_pipeline(
            body,
            grid=(N // SCL,),
            in_specs=[pl.BlockSpec((H * E, SCL), lambda i: (0, i))],
            out_specs=[pl.BlockSpec((H * E, SCL), lambda i: (0, i))],
            core_axis_name=("core", "subcore"),
            dimension_semantics=(pltpu.PARALLEL,),
        )(l_hbm, o_hbm)

    return run(logitsT)


def _expert_block(h_ref, wt_ref, w_up_ref, w_down_ref, w_opt_ref, b_op_ref,
                  y_ref):
    hb = h_ref[...]
    w = wt_ref[...].T.astype(jnp.bfloat16)
    dns = []
    for hh in range(H):
        hsb = hb[:, hh * HD:(hh + 1) * HD]
        up = jnp.dot(hsb, w_up_ref[...],
                     preferred_element_type=jnp.float32
                     ).astype(jnp.bfloat16)
        parts = []
        for e in range(E):
            ue = up[:, e * I:(e + 1) * I]
            ae = (jnp.square(jnp.maximum(ue, 0.0))
                  * w[:, hh * E + e:hh * E + e + 1])
            parts.append(ae)
        a = jnp.concatenate(parts, axis=1)
        dn = jnp.dot(a, w_down_ref[...],
                     preferred_element_type=jnp.float32)
        dns.append(dn.astype(jnp.bfloat16))
    dn_all = jnp.concatenate(dns, axis=1)
    y = jnp.dot(dn_all, w_opt_ref[...], preferred_element_type=jnp.float32)
    y_ref[...] = y + b_op_ref[...]


@jax.jit
def kernel(x, W_hp, b_hp, expert_emb, W_up, W_down, W_op, b_op):
    W_hpT = W_hp.T.astype(jnp.bfloat16)
    embbd = block_diag(*([expert_emb.T] * H))
    W_up_r = W_up.transpose(1, 0, 2).reshape(HD, E * I).astype(jnp.bfloat16)
    W_down_r = W_down.reshape(E * I, HD).astype(jnp.bfloat16)
    W_opT = W_op.T.astype(jnp.bfloat16)
    b_hp2 = b_hp.reshape(1, D)
    b_op2 = b_op.reshape(1, D)

    h_bf, logitsT = pl.pallas_call(
        _head_router_block,
        grid=(N // TB,),
        in_specs=[
            pl.BlockSpec((TB, D), lambda i: (i, 0)),
            pl.BlockSpec((D, D), lambda i: (0, 0)),
            pl.BlockSpec((1, D), lambda i: (0, 0)),
            pl.BlockSpec((D, H * E), lambda i: (0, 0)),
        ],
        out_specs=[
            pl.BlockSpec((TB, D), lambda i: (i, 0)),
            pl.BlockSpec((H * E, TB), lambda i: (0, i)),
        ],
        out_shape=[
            jax.ShapeDtypeStruct((N, D), jnp.bfloat16),
            jax.ShapeDtypeStruct((H * E, N), jnp.float32),
        ],
        compiler_params=pltpu.CompilerParams(
            dimension_semantics=("parallel",)),
    )(x, W_hpT, b_hp2, embbd)

    w64 = _router_sc(logitsT)

    return pl.pallas_call(
        _expert_block,
        grid=(N // TB,),
        in_specs=[
            pl.BlockSpec((TB, D), lambda i: (i, 0)),
            pl.BlockSpec((H * E, TB), lambda i: (0, i)),
            pl.BlockSpec((HD, E * I), lambda i: (0, 0)),
            pl.BlockSpec((E * I, HD), lambda i: (0, 0)),
            pl.BlockSpec((D, D), lambda i: (0, 0)),
            pl.BlockSpec((1, D), lambda i: (0, 0)),
        ],
        out_specs=pl.BlockSpec((TB, D), lambda i: (i, 0)),
        out_shape=jax.ShapeDtypeStruct((N, D), jnp.float32),
        compiler_params=pltpu.CompilerParams(
            dimension_semantics=("parallel",)),
    )(h_bf, w64, W_up_r, W_down_r, W_opT, b_op2)
